# baseline (speedup 1.0000x reference)
import functools

import jax
import jax.numpy as jnp
from jax import lax
from jax.experimental import pallas as pl
from jax.experimental.pallas import tpu as pltpu
from jax.experimental.pallas import tpu_sc as plsc

_BATCH = 4
_SEQ = 8192
_DIM = 1024
_NW = 32
_S_PER_W = _SEQ // _NW
_SB = 16
_NSB = _S_PER_W // _SB
_LANES = 16
_UNROLL = 8
_DPL = _DIM // _LANES
_DPL_SHIFT = 6


def _sc_body(x_hbm, pos_hbm, out_hbm, pos_buf, x_buf):
    wid = lax.axis_index("s") * 2 + lax.axis_index("c")
    base = wid * _S_PER_W

    def sb_loop(i, carry):
        pos0 = base + i * _SB
        pltpu.sync_copy(pos_hbm.at[pl.ds(pos0, _SB)], pos_buf)
        for b in range(_BATCH):
            pltpu.sync_copy(x_hbm.at[b, pl.ds(pos0, _SB)], x_buf.at[b])

        @plsc.parallel_loop(0, _SB * _DIM // _LANES, 1, unroll=_UNROLL)
        def _add(v):
            r = lax.shift_right_logical(v, _DPL_SHIFT)
            col = pl.multiple_of(
                lax.shift_left(lax.bitwise_and(v, _DPL - 1), 4), _LANES
            )
            sl = pl.ds(col, _LANES)
            p = pos_buf[r, sl]
            for b in range(_BATCH):
                x_buf[b, r, sl] = x_buf[b, r, sl] + p
        for b in range(_BATCH):
            pltpu.sync_copy(x_buf.at[b], out_hbm.at[b, pl.ds(pos0, _SB)])
        return carry

    lax.fori_loop(0, _NSB, sb_loop, 0)


def kernel(x, pos_table):
    mesh = plsc.VectorSubcoreMesh(core_axis_name="c", subcore_axis_name="s")
    run = functools.partial(
        pl.kernel,
        mesh=mesh,
        out_type=jax.ShapeDtypeStruct((_BATCH, _SEQ, _DIM), jnp.float32),
        scratch_types=[
            pltpu.VMEM((_SB, _DIM), jnp.float32),
            pltpu.VMEM((_BATCH, _SB, _DIM), jnp.float32),
        ],
    )(_sc_body)
    return run(x, pos_table)


# speedup vs baseline: 1.7024x; 1.7024x over previous
import functools

import jax
import jax.numpy as jnp
from jax import lax
from jax.experimental import pallas as pl
from jax.experimental.pallas import tpu as pltpu
from jax.experimental.pallas import tpu_sc as plsc

_BATCH = 4
_SEQ = 8192
_DIM = 1024
_NW = 32
_S_PER_W = _SEQ // _NW
_SB = 8
_NSB = _S_PER_W // _SB
_LANES = 16
_UNROLL = 8
_DPL = _DIM // _LANES
_DPL_SHIFT = 6


def _sc_body(x_hbm, pos_hbm, out_hbm, pos_buf, x_buf,
             in_sem0, in_sem1, out_sem0, out_sem1):
    wid = lax.axis_index("s") * 2 + lax.axis_index("c")
    base = wid * _S_PER_W
    in_sems = (in_sem0, in_sem1)
    out_sems = (out_sem0, out_sem1)

    def in_copies(ch, slot):
        pos0 = base + ch * _SB
        yield pltpu.make_async_copy(
            pos_hbm.at[pl.ds(pos0, _SB)], pos_buf.at[slot], in_sems[slot])
        for b in range(_BATCH):
            yield pltpu.make_async_copy(
                x_hbm.at[b, pl.ds(pos0, _SB)], x_buf.at[slot, b],
                in_sems[slot])

    def out_copies(ch, slot):
        pos0 = base + ch * _SB
        for b in range(_BATCH):
            yield pltpu.make_async_copy(
                x_buf.at[slot, b], out_hbm.at[b, pl.ds(pos0, _SB)],
                out_sems[slot])

    def start_in(ch, slot):
        for c in in_copies(ch, slot):
            c.start()

    def wait_in(ch, slot):
        for c in in_copies(ch, slot):
            c.wait()

    def start_out(ch, slot):
        for c in out_copies(ch, slot):
            c.start()

    def wait_out(ch, slot):
        for c in out_copies(ch, slot):
            c.wait()

    def compute(slot):
        @plsc.parallel_loop(0, _SB * _DPL, 1, unroll=_UNROLL)
        def _add(v):
            r = lax.shift_right_logical(v, _DPL_SHIFT)
            col = pl.multiple_of(
                lax.shift_left(lax.bitwise_and(v, _DPL - 1), 4), _LANES)
            sl = pl.ds(col, _LANES)
            p = pos_buf[slot, r, sl]
            for b in range(_BATCH):
                x_buf[slot, b, r, sl] = x_buf[slot, b, r, sl] + p

    def half(i, s_cur, s_next):
        @pl.when(i + 1 < _NSB)
        def _():
            @pl.when(i >= 1)
            def _():
                wait_out(i - 1, s_next)
            start_in(i + 1, s_next)

        wait_in(i, s_cur)
        compute(s_cur)
        start_out(i, s_cur)

    start_in(0, 0)

    def pair_loop(ci, carry):
        half(ci * 2, 0, 1)
        half(ci * 2 + 1, 1, 0)
        return carry

    lax.fori_loop(0, _NSB // 2, pair_loop, 0)
    wait_out(_NSB - 2, 0)
    wait_out(_NSB - 1, 1)


def kernel(x, pos_table):
    mesh = plsc.VectorSubcoreMesh(core_axis_name="c", subcore_axis_name="s")
    run = functools.partial(
        pl.kernel,
        mesh=mesh,
        out_type=jax.ShapeDtypeStruct((_BATCH, _SEQ, _DIM), jnp.float32),
        scratch_types=[
            pltpu.VMEM((2, _SB, _DIM), jnp.float32),
            pltpu.VMEM((2, _BATCH, _SB, _DIM), jnp.float32),
            pltpu.SemaphoreType.DMA,
            pltpu.SemaphoreType.DMA,
            pltpu.SemaphoreType.DMA,
            pltpu.SemaphoreType.DMA,
        ],
    )(_sc_body)
    return run(x, pos_table)
